# Initial kernel scaffold; baseline (speedup 1.0000x reference)
import jax
import jax.numpy as jnp
from jax.experimental import pallas as pl


def kernel(x_H, x_S, edge_index, W_q, W_k, W_v, W_s, W_p):
    raise NotImplementedError("write your pallas kernel here")


# speedup vs baseline: 13.8358x; 13.8358x over previous
import functools

import jax
import jax.numpy as jnp
from jax import lax
from jax.experimental import pallas as pl
from jax.experimental.pallas import tpu as pltpu
from jax.experimental.pallas import tpu_sc as plsc

N = 10000
E = 320000
D = 128
NP = 10240
NCORES = 2
NSUB = 16
NW = NCORES * NSUB
EPT = E // NW
C = 80
NCHUNK = EPT // C
ROWS_PER_TILE = NP // NSUB


def _lorentz_sq(x):
    full = jnp.sum(x * x, axis=-1, keepdims=True)
    return full - 2.0 * x[:, :1] * x[:, :1]


def _norm_factor(x):
    return jnp.sqrt(jnp.clip(jnp.abs(_lorentz_sq(x)), 1e-8, None))


def _pre_body(xs_ref, xh_ref, wq_ref, wk_ref, wv_ref, ws_ref,
              a_ref, b_ref, v_ref, bm_ref):
    xs = xs_ref[...]
    xh = xh_ref[...]
    wq = wq_ref[...]
    wk = wk_ref[...]
    wv = wv_ref[...]
    ws = ws_ref[...]

    qp = jnp.dot(xs, wq.T, preferred_element_type=jnp.float32)
    kp = jnp.dot(xh, wk.T, preferred_element_type=jnp.float32)
    vp = jnp.dot(xh, wv.T, preferred_element_type=jnp.float32)

    q = qp / _norm_factor(qp)
    k = kp / _norm_factor(kp)
    v = vp / _norm_factor(vp)

    w1 = ws[0:1, 0:D]
    w2 = ws[0:1, D:2 * D]
    a_ref[...] = jnp.sum(q * w1, axis=-1, keepdims=True)
    bcol = jnp.sum(k * w2, axis=-1, keepdims=True)
    b_ref[...] = bcol
    v_ref[...] = v

    bb = jnp.max(bcol).reshape(1, 1)

    @pl.when(pl.program_id(0) == 0)
    def _():
        bm_ref[...] = bb

    @pl.when(pl.program_id(0) > 0)
    def _():
        bm_ref[...] = jnp.maximum(bm_ref[...], bb)


def _pre(x_S, x_H, W_q, W_k, W_v, W_s):
    blk = 1000
    grid = N // blk
    return pl.pallas_call(
        _pre_body,
        grid=(grid,),
        in_specs=[
            pl.BlockSpec((blk, D), lambda i: (i, 0)),
            pl.BlockSpec((blk, D), lambda i: (i, 0)),
            pl.BlockSpec((D, D), lambda i: (0, 0)),
            pl.BlockSpec((D, D), lambda i: (0, 0)),
            pl.BlockSpec((D, D), lambda i: (0, 0)),
            pl.BlockSpec((1, 2 * D), lambda i: (0, 0)),
        ],
        out_specs=[
            pl.BlockSpec((blk, 1), lambda i: (i, 0)),
            pl.BlockSpec((blk, 1), lambda i: (i, 0)),
            pl.BlockSpec((blk, D), lambda i: (i, 0)),
            pl.BlockSpec((1, 1), lambda i: (0, 0)),
        ],
        out_shape=[
            jax.ShapeDtypeStruct((N, 1), jnp.float32),
            jax.ShapeDtypeStruct((N, 1), jnp.float32),
            jax.ShapeDtypeStruct((N, D), jnp.float32),
            jax.ShapeDtypeStruct((1, 1), jnp.float32),
        ],
    )(x_S, x_H, W_q, W_k, W_v, W_s)


def _sc_body(a_hbm, b_hbm, bm_hbm, v_hbm, src_hbm, dst_hbm,
             numer_hbm, denom_hbm,
             at, bt, bmv, sidx, didx, exb, rows, acc, nsh, dsh, sem):
    cid = lax.axis_index("c")
    sid = lax.axis_index("s")
    wid = cid * NSUB + sid

    zero16 = jnp.zeros((16,), jnp.float32)

    pltpu.sync_copy(a_hbm, at)
    pltpu.sync_copy(b_hbm, bt)
    pltpu.sync_copy(bm_hbm, bmv)

    def _zrow(r, _):
        for j in range(8):
            rows[r, pl.ds(j * 16, 16)] = zero16
        return 0
    lax.fori_loop(0, C, _zrow, 0)

    def _zacc(i, _):
        acc[pl.ds(i * 16, 16)] = zero16
        return 0
    lax.fori_loop(0, ROWS_PER_TILE // 16, _zacc, 0)

    nbase = sid * ROWS_PER_TILE
    for t in range(ROWS_PER_TILE // C):
        pltpu.sync_copy(rows, nsh.at[pl.ds(nbase + t * C, C)])
    pltpu.sync_copy(acc, dsh.at[pl.ds(nbase, ROWS_PER_TILE)])
    plsc.subcore_barrier()

    bmax = bmv[pl.ds(0, 16)]

    ebase = wid * EPT

    def _echunk(i, _):
        off = ebase + i * C
        pltpu.sync_copy(src_hbm.at[pl.ds(off, C)], sidx)
        pltpu.sync_copy(dst_hbm.at[pl.ds(off, C)], didx)
        pltpu.async_copy(v_hbm.at[didx], rows, sem).wait()
        for g in range(C // 16):
            s16 = sidx[pl.ds(g * 16, 16)]
            d16 = didx[pl.ds(g * 16, 16)]
            av = plsc.load_gather(at, [s16])
            bv = plsc.load_gather(bt, [d16])
            x = av + bv
            sc = jnp.maximum(x, 0.01 * x)
            xm = av + bmax
            cm = jnp.maximum(xm, 0.01 * xm)
            ex = jnp.exp(sc - cm)
            exb[pl.ds(g * 16, 16)] = ex

        def _scale(r, _):
            w = plsc.load_gather(exb, [jnp.broadcast_to(r, (16,))])
            for j in range(8):
                rows[r, pl.ds(j * 16, 16)] = rows[r, pl.ds(j * 16, 16)] * w
            return 0
        lax.fori_loop(0, C, _scale, 0)

        pltpu.sync_copy(rows, nsh.at[sidx], add=True)
        pltpu.sync_copy(exb, dsh.at[sidx], add=True)
        return 0

    lax.fori_loop(0, NCHUNK, _echunk, 0)
    plsc.subcore_barrier()

    pltpu.sync_copy(dsh.at[pl.ds(nbase, ROWS_PER_TILE)],
                    denom_hbm.at[cid, pl.ds(nbase, ROWS_PER_TILE)])
    pltpu.sync_copy(nsh.at[pl.ds(nbase, ROWS_PER_TILE)],
                    numer_hbm.at[cid, pl.ds(nbase, ROWS_PER_TILE)])


@functools.partial(
    pl.kernel,
    out_type=[
        jax.ShapeDtypeStruct((NCORES, NP, D), jnp.float32),
        jax.ShapeDtypeStruct((NCORES, NP), jnp.float32),
    ],
    mesh=plsc.VectorSubcoreMesh(core_axis_name="c", subcore_axis_name="s"),
    compiler_params=pltpu.CompilerParams(needs_layout_passes=False),
    scratch_types=[
        pltpu.VMEM((N,), jnp.float32),
        pltpu.VMEM((N,), jnp.float32),
        pltpu.VMEM((16,), jnp.float32),
        pltpu.VMEM((C,), jnp.int32),
        pltpu.VMEM((C,), jnp.int32),
        pltpu.VMEM((C,), jnp.float32),
        pltpu.VMEM((C, D), jnp.float32),
        pltpu.VMEM((ROWS_PER_TILE,), jnp.float32),
        pltpu.VMEM_SHARED((NP, D), jnp.float32),
        pltpu.VMEM_SHARED((NP,), jnp.float32),
        pltpu.SemaphoreType.DMA,
    ],
)
def _edge_sc(a_hbm, b_hbm, bm_hbm, v_hbm, src_hbm, dst_hbm,
             numer_hbm, denom_hbm,
             at, bt, bmv, sidx, didx, exb, rows, acc, nsh, dsh, sem):
    _sc_body(a_hbm, b_hbm, bm_hbm, v_hbm, src_hbm, dst_hbm,
             numer_hbm, denom_hbm,
             at, bt, bmv, sidx, didx, exb, rows, acc, nsh, dsh, sem)


def _post_body(n0_ref, n1_ref, d0_ref, d1_ref, xh_ref, wp_ref, z_ref):
    agg = n0_ref[...] + n1_ref[...]
    d = d0_ref[...] + d1_ref[...]
    inv = jnp.where(d > 0.0, 1.0 / d, 0.0)
    out = agg * inv
    col = lax.broadcasted_iota(jnp.int32, out.shape, 1)
    out = out + jnp.where(col == 0, 1.0, 0.0)
    out = out / _norm_factor(out)
    op = jnp.dot(out, wp_ref[...].T, preferred_element_type=jnp.float32)
    op = op / _norm_factor(op)
    s = op + xh_ref[...]
    z_ref[...] = s / _norm_factor(s)


def _post(n0, n1, d0, d1, x_H, W_p):
    blk = 1000
    grid = N // blk
    return pl.pallas_call(
        _post_body,
        grid=(grid,),
        in_specs=[
            pl.BlockSpec((blk, D), lambda i: (i, 0)),
            pl.BlockSpec((blk, D), lambda i: (i, 0)),
            pl.BlockSpec((blk, 1), lambda i: (i, 0)),
            pl.BlockSpec((blk, 1), lambda i: (i, 0)),
            pl.BlockSpec((blk, D), lambda i: (i, 0)),
            pl.BlockSpec((D, D), lambda i: (0, 0)),
        ],
        out_specs=pl.BlockSpec((blk, D), lambda i: (i, 0)),
        out_shape=jax.ShapeDtypeStruct((N, D), jnp.float32),
    )(n0, n1, d0, d1, x_H, W_p)


@jax.jit
def kernel(x_H, x_S, edge_index, W_q, W_k, W_v, W_s, W_p):
    src = edge_index[0].astype(jnp.int32)
    dst = edge_index[1].astype(jnp.int32)
    a2, b2, v, bm = _pre(x_S, x_H, W_q, W_k, W_v, W_s)
    bvec = jnp.broadcast_to(bm[0], (16,))
    numer, denom = _edge_sc(a2[:, 0], b2[:, 0], bvec, v, src, dst)
    z = _post(numer[0, :N], numer[1, :N],
              denom[0, :N, None], denom[1, :N, None], x_H, W_p)
    return z


# speedup vs baseline: 18.6581x; 1.3485x over previous
import functools

import jax
import jax.numpy as jnp
from jax import lax
from jax.experimental import pallas as pl
from jax.experimental.pallas import tpu as pltpu
from jax.experimental.pallas import tpu_sc as plsc

N = 10000
E = 320000
D = 128
NP = 10240
NCORES = 2
NSUB = 16
NW = NCORES * NSUB
EPT = E // NW
C = 80
NCHUNK = EPT // C
ROWS_PER_TILE = NP // NSUB


def _lorentz_sq(x):
    full = jnp.sum(x * x, axis=-1, keepdims=True)
    return full - 2.0 * x[:, :1] * x[:, :1]


def _norm_factor(x):
    return jnp.sqrt(jnp.clip(jnp.abs(_lorentz_sq(x)), 1e-8, None))


def _pre_body(xs_ref, xh_ref, wq_ref, wk_ref, wv_ref, ws_ref,
              a_ref, b_ref, v_ref, bm_ref):
    xs = xs_ref[...]
    xh = xh_ref[...]
    wq = wq_ref[...]
    wk = wk_ref[...]
    wv = wv_ref[...]
    ws = ws_ref[...]

    qp = jnp.dot(xs, wq.T, preferred_element_type=jnp.float32)
    kp = jnp.dot(xh, wk.T, preferred_element_type=jnp.float32)
    vp = jnp.dot(xh, wv.T, preferred_element_type=jnp.float32)

    q = qp / _norm_factor(qp)
    k = kp / _norm_factor(kp)
    v = vp / _norm_factor(vp)

    w1 = ws[0:1, 0:D]
    w2 = ws[0:1, D:2 * D]
    a_ref[...] = jnp.sum(q * w1, axis=-1, keepdims=True)
    bcol = jnp.sum(k * w2, axis=-1, keepdims=True)
    b_ref[...] = bcol
    v_ref[...] = v

    bb = jnp.max(bcol).reshape(1, 1)

    @pl.when(pl.program_id(0) == 0)
    def _():
        bm_ref[...] = bb

    @pl.when(pl.program_id(0) > 0)
    def _():
        bm_ref[...] = jnp.maximum(bm_ref[...], bb)


def _pre(x_S, x_H, W_q, W_k, W_v, W_s):
    blk = 1000
    grid = N // blk
    return pl.pallas_call(
        _pre_body,
        grid=(grid,),
        in_specs=[
            pl.BlockSpec((blk, D), lambda i: (i, 0)),
            pl.BlockSpec((blk, D), lambda i: (i, 0)),
            pl.BlockSpec((D, D), lambda i: (0, 0)),
            pl.BlockSpec((D, D), lambda i: (0, 0)),
            pl.BlockSpec((D, D), lambda i: (0, 0)),
            pl.BlockSpec((1, 2 * D), lambda i: (0, 0)),
        ],
        out_specs=[
            pl.BlockSpec((blk, 1), lambda i: (i, 0)),
            pl.BlockSpec((blk, 1), lambda i: (i, 0)),
            pl.BlockSpec((blk, D), lambda i: (i, 0)),
            pl.BlockSpec((1, 1), lambda i: (0, 0)),
        ],
        out_shape=[
            jax.ShapeDtypeStruct((N, 1), jnp.float32),
            jax.ShapeDtypeStruct((N, 1), jnp.float32),
            jax.ShapeDtypeStruct((N, D), jnp.float32),
            jax.ShapeDtypeStruct((1, 1), jnp.float32),
        ],
    )(x_S, x_H, W_q, W_k, W_v, W_s)


def _sc_body(a_hbm, b_hbm, bm_hbm, v_hbm, src_hbm, dst_hbm,
             numer_hbm, denom_hbm,
             at, bt, bmv, sidx0, didx0, exb0, rows0,
             sidx1, didx1, exb1, rows1, acc, nsh, dsh, sem0, sem1):
    cid = lax.axis_index("c")
    sid = lax.axis_index("s")
    wid = cid * NSUB + sid

    zero16 = jnp.zeros((16,), jnp.float32)

    pltpu.sync_copy(a_hbm, at)
    pltpu.sync_copy(b_hbm, bt)
    pltpu.sync_copy(bm_hbm, bmv)

    def _zrow(r, _):
        for j in range(8):
            rows0[r, pl.ds(j * 16, 16)] = zero16
        return 0
    lax.fori_loop(0, C, _zrow, 0)

    def _zacc(i, _):
        acc[pl.ds(i * 16, 16)] = zero16
        return 0
    lax.fori_loop(0, ROWS_PER_TILE // 16, _zacc, 0)

    nbase = sid * ROWS_PER_TILE
    for t in range(ROWS_PER_TILE // C):
        pltpu.sync_copy(rows0, nsh.at[pl.ds(nbase + t * C, C)])
    pltpu.sync_copy(acc, dsh.at[pl.ds(nbase, ROWS_PER_TILE)])
    plsc.subcore_barrier()

    bmax = bmv[pl.ds(0, 16)]

    ebase = wid * EPT
    bufs = [(sidx0, didx0, exb0, rows0, sem0),
            (sidx1, didx1, exb1, rows1, sem1)]

    def _prefetch(i, buf):
        sidx_b, didx_b, _, rows_b, sem_b = buf
        off = ebase + i * C
        pltpu.sync_copy(src_hbm.at[pl.ds(off, C)], sidx_b)
        pltpu.sync_copy(dst_hbm.at[pl.ds(off, C)], didx_b)
        pltpu.async_copy(v_hbm.at[didx_b], rows_b, sem_b)

    def _process(buf):
        sidx_b, didx_b, exb_b, rows_b, sem_b = buf
        pltpu.make_async_copy(v_hbm.at[didx_b], rows_b, sem_b).wait()
        for g in range(C // 16):
            s16 = sidx_b[pl.ds(g * 16, 16)]
            d16 = didx_b[pl.ds(g * 16, 16)]
            av = plsc.load_gather(at, [s16])
            bv = plsc.load_gather(bt, [d16])
            x = av + bv
            sc = jnp.maximum(x, 0.01 * x)
            xm = av + bmax
            cm = jnp.maximum(xm, 0.01 * xm)
            ex = jnp.exp(sc - cm)
            exb_b[pl.ds(g * 16, 16)] = ex

        def _scale(r4, _):
            for u in range(4):
                r = r4 * 4 + u
                w = plsc.load_gather(exb_b, [jnp.broadcast_to(r, (16,))])
                for j in range(8):
                    rows_b[r, pl.ds(j * 16, 16)] = (
                        rows_b[r, pl.ds(j * 16, 16)] * w)
            return 0
        lax.fori_loop(0, C // 4, _scale, 0)

        pltpu.sync_copy(rows_b, nsh.at[sidx_b], add=True)
        pltpu.sync_copy(exb_b, dsh.at[sidx_b], add=True)

    _prefetch(0, bufs[0])
    _prefetch(1, bufs[1])

    def _outer(io, _):
        for b in range(2):
            i = io * 2 + b
            _process(bufs[b])

            @pl.when(i + 2 < NCHUNK)
            def _():
                _prefetch(i + 2, bufs[b])
        return 0

    lax.fori_loop(0, (NCHUNK - 1) // 2, _outer, 0)
    _process(bufs[(NCHUNK - 1) % 2])
    plsc.subcore_barrier()

    pltpu.sync_copy(dsh.at[pl.ds(nbase, ROWS_PER_TILE)],
                    denom_hbm.at[cid, pl.ds(nbase, ROWS_PER_TILE)])
    pltpu.sync_copy(nsh.at[pl.ds(nbase, ROWS_PER_TILE)],
                    numer_hbm.at[cid, pl.ds(nbase, ROWS_PER_TILE)])


@functools.partial(
    pl.kernel,
    out_type=[
        jax.ShapeDtypeStruct((NCORES, NP, D), jnp.float32),
        jax.ShapeDtypeStruct((NCORES, NP), jnp.float32),
    ],
    mesh=plsc.VectorSubcoreMesh(core_axis_name="c", subcore_axis_name="s"),
    compiler_params=pltpu.CompilerParams(needs_layout_passes=False),
    scratch_types=[
        pltpu.VMEM((N,), jnp.float32),
        pltpu.VMEM((N,), jnp.float32),
        pltpu.VMEM((16,), jnp.float32),
        pltpu.VMEM((C,), jnp.int32),
        pltpu.VMEM((C,), jnp.int32),
        pltpu.VMEM((C,), jnp.float32),
        pltpu.VMEM((C, D), jnp.float32),
        pltpu.VMEM((C,), jnp.int32),
        pltpu.VMEM((C,), jnp.int32),
        pltpu.VMEM((C,), jnp.float32),
        pltpu.VMEM((C, D), jnp.float32),
        pltpu.VMEM((ROWS_PER_TILE,), jnp.float32),
        pltpu.VMEM_SHARED((NP, D), jnp.float32),
        pltpu.VMEM_SHARED((NP,), jnp.float32),
        pltpu.SemaphoreType.DMA,
        pltpu.SemaphoreType.DMA,
    ],
)
def _edge_sc(a_hbm, b_hbm, bm_hbm, v_hbm, src_hbm, dst_hbm,
             numer_hbm, denom_hbm,
             at, bt, bmv, sidx0, didx0, exb0, rows0,
             sidx1, didx1, exb1, rows1, acc, nsh, dsh, sem0, sem1):
    _sc_body(a_hbm, b_hbm, bm_hbm, v_hbm, src_hbm, dst_hbm,
             numer_hbm, denom_hbm,
             at, bt, bmv, sidx0, didx0, exb0, rows0,
             sidx1, didx1, exb1, rows1, acc, nsh, dsh, sem0, sem1)


def _post_body(n0_ref, n1_ref, d0_ref, d1_ref, xh_ref, wp_ref, z_ref):
    agg = n0_ref[...] + n1_ref[...]
    d = d0_ref[...] + d1_ref[...]
    inv = jnp.where(d > 0.0, 1.0 / d, 0.0)
    out = agg * inv
    col = lax.broadcasted_iota(jnp.int32, out.shape, 1)
    out = out + jnp.where(col == 0, 1.0, 0.0)
    out = out / _norm_factor(out)
    op = jnp.dot(out, wp_ref[...].T, preferred_element_type=jnp.float32)
    op = op / _norm_factor(op)
    s = op + xh_ref[...]
    z_ref[...] = s / _norm_factor(s)


def _post(n0, n1, d0, d1, x_H, W_p):
    blk = 1000
    grid = N // blk
    return pl.pallas_call(
        _post_body,
        grid=(grid,),
        in_specs=[
            pl.BlockSpec((blk, D), lambda i: (i, 0)),
            pl.BlockSpec((blk, D), lambda i: (i, 0)),
            pl.BlockSpec((blk, 1), lambda i: (i, 0)),
            pl.BlockSpec((blk, 1), lambda i: (i, 0)),
            pl.BlockSpec((blk, D), lambda i: (i, 0)),
            pl.BlockSpec((D, D), lambda i: (0, 0)),
        ],
        out_specs=pl.BlockSpec((blk, D), lambda i: (i, 0)),
        out_shape=jax.ShapeDtypeStruct((N, D), jnp.float32),
    )(n0, n1, d0, d1, x_H, W_p)


@jax.jit
def kernel(x_H, x_S, edge_index, W_q, W_k, W_v, W_s, W_p):
    src = edge_index[0].astype(jnp.int32)
    dst = edge_index[1].astype(jnp.int32)
    a2, b2, v, bm = _pre(x_S, x_H, W_q, W_k, W_v, W_s)
    bvec = jnp.broadcast_to(bm[0], (16,))
    numer, denom = _edge_sc(a2[:, 0], b2[:, 0], bvec, v, src, dst)
    z = _post(numer[0, :N], numer[1, :N],
              denom[0, :N, None], denom[1, :N, None], x_H, W_p)
    return z


# speedup vs baseline: 24.8315x; 1.3309x over previous
import functools

import jax
import jax.numpy as jnp
from jax import lax
from jax.experimental import pallas as pl
from jax.experimental.pallas import tpu as pltpu
from jax.experimental.pallas import tpu_sc as plsc

N = 10000
E = 320000
D = 128
NP = 10240
NCORES = 2
NSUB = 16
NW = NCORES * NSUB
EPT = E // NW
C = 80
NCHUNK = EPT // C
ROWS_PER_TILE = NP // NSUB


def _lorentz_sq(x):
    full = jnp.sum(x * x, axis=-1, keepdims=True)
    return full - 2.0 * x[:, :1] * x[:, :1]


def _norm_factor(x):
    return jnp.sqrt(jnp.clip(jnp.abs(_lorentz_sq(x)), 1e-8, None))


def _pre_body(xs_ref, xh_ref, wq_ref, wk_ref, wv_ref, ws_ref,
              a_ref, b_ref, v_ref, bm_ref):
    xs = xs_ref[...]
    xh = xh_ref[...]
    wq = wq_ref[...]
    wk = wk_ref[...]
    wv = wv_ref[...]
    ws = ws_ref[...]

    qp = jnp.dot(xs, wq.T, preferred_element_type=jnp.float32)
    kp = jnp.dot(xh, wk.T, preferred_element_type=jnp.float32)
    vp = jnp.dot(xh, wv.T, preferred_element_type=jnp.float32)

    q = qp / _norm_factor(qp)
    k = kp / _norm_factor(kp)
    v = vp / _norm_factor(vp)

    w1 = ws[0:1, 0:D]
    w2 = ws[0:1, D:2 * D]
    a_ref[...] = jnp.sum(q * w1, axis=-1, keepdims=True)
    bcol = jnp.sum(k * w2, axis=-1, keepdims=True)
    b_ref[...] = bcol
    v_ref[...] = v

    bb = jnp.max(bcol).reshape(1, 1)

    @pl.when(pl.program_id(0) == 0)
    def _():
        bm_ref[...] = bb

    @pl.when(pl.program_id(0) > 0)
    def _():
        bm_ref[...] = jnp.maximum(bm_ref[...], bb)


def _pre(x_S, x_H, W_q, W_k, W_v, W_s):
    blk = 1000
    grid = N // blk
    return pl.pallas_call(
        _pre_body,
        grid=(grid,),
        in_specs=[
            pl.BlockSpec((blk, D), lambda i: (i, 0)),
            pl.BlockSpec((blk, D), lambda i: (i, 0)),
            pl.BlockSpec((D, D), lambda i: (0, 0)),
            pl.BlockSpec((D, D), lambda i: (0, 0)),
            pl.BlockSpec((D, D), lambda i: (0, 0)),
            pl.BlockSpec((1, 2 * D), lambda i: (0, 0)),
        ],
        out_specs=[
            pl.BlockSpec((blk, 1), lambda i: (i, 0)),
            pl.BlockSpec((blk, 1), lambda i: (i, 0)),
            pl.BlockSpec((blk, D), lambda i: (i, 0)),
            pl.BlockSpec((1, 1), lambda i: (0, 0)),
        ],
        out_shape=[
            jax.ShapeDtypeStruct((N, 1), jnp.float32),
            jax.ShapeDtypeStruct((N, 1), jnp.float32),
            jax.ShapeDtypeStruct((N, D), jnp.float32),
            jax.ShapeDtypeStruct((1, 1), jnp.float32),
        ],
    )(x_S, x_H, W_q, W_k, W_v, W_s)


def _sc_body(a_hbm, b_hbm, bm_hbm, v_hbm, ed_hbm,
             numer_hbm, denom_hbm,
             at, bt, bmv, idxp0, idxp1, idxp2, idxp3,
             exb0, rows0, exb1, rows1,
             acc, nsh, dsh, semi0, semi1, semi2, semi3, semg0, semg1):
    cid = lax.axis_index("c")
    sid = lax.axis_index("s")
    wid = cid * NSUB + sid

    zero16 = jnp.zeros((16,), jnp.float32)

    pltpu.sync_copy(a_hbm, at)
    pltpu.sync_copy(b_hbm, bt)
    pltpu.sync_copy(bm_hbm, bmv)

    def _zrow(r, _):
        for j in range(8):
            rows0[r, pl.ds(j * 16, 16)] = zero16
        return 0
    lax.fori_loop(0, C, _zrow, 0)

    def _zacc(i, _):
        acc[pl.ds(i * 16, 16)] = zero16
        return 0
    lax.fori_loop(0, ROWS_PER_TILE // 16, _zacc, 0)

    nbase = sid * ROWS_PER_TILE
    for t in range(ROWS_PER_TILE // C):
        pltpu.sync_copy(rows0, nsh.at[pl.ds(nbase + t * C, C)])
    pltpu.sync_copy(acc, dsh.at[pl.ds(nbase, ROWS_PER_TILE)])
    plsc.subcore_barrier()

    bmax = bmv[pl.ds(0, 16)]

    cbase = wid * NCHUNK
    ibufs = [(idxp0, semi0), (idxp1, semi1), (idxp2, semi2), (idxp3, semi3)]
    rbufs = [(exb0, rows0, semg0), (exb1, rows1, semg1)]

    def _idx_start(i, bi):
        idx_b, semi_b = ibufs[bi]
        pltpu.async_copy(ed_hbm.at[cbase + i], idx_b, semi_b)

    def _idx_wait(bi):
        idx_b, semi_b = ibufs[bi]
        pltpu.make_async_copy(ed_hbm.at[cbase], idx_b, semi_b).wait()

    def _gather(bi, br):
        idx_b, _ = ibufs[bi]
        _, rows_b, semg_b = rbufs[br]
        pltpu.async_copy(v_hbm.at[idx_b.at[1]], rows_b, semg_b)

    def _step(i, bi, br):
        idx_b, _ = ibufs[bi]
        exb_b, rows_b, semg_b = rbufs[br]
        pltpu.make_async_copy(v_hbm.at[idx_b.at[1]], rows_b, semg_b).wait()
        for g in range(C // 16):
            s16 = idx_b[0, pl.ds(g * 16, 16)]
            d16 = idx_b[1, pl.ds(g * 16, 16)]
            av = plsc.load_gather(at, [s16])
            bv = plsc.load_gather(bt, [d16])
            x = av + bv
            sc = jnp.maximum(x, 0.01 * x)
            xm = av + bmax
            cm = jnp.maximum(xm, 0.01 * xm)
            ex = jnp.exp(sc - cm)
            exb_b[pl.ds(g * 16, 16)] = ex

        def _scale(r4, _):
            for u in range(4):
                r = r4 * 4 + u
                w = plsc.load_gather(exb_b, [jnp.broadcast_to(r, (16,))])
                for j in range(8):
                    rows_b[r, pl.ds(j * 16, 16)] = (
                        rows_b[r, pl.ds(j * 16, 16)] * w)
            return 0
        lax.fori_loop(0, C // 4, _scale, 0)

        pltpu.sync_copy(rows_b, nsh.at[idx_b.at[0]], add=True)
        pltpu.sync_copy(exb_b, dsh.at[idx_b.at[0]], add=True)

        @pl.when(i + 2 < NCHUNK)
        def _():
            _idx_wait((bi + 2) % 4)
            _gather((bi + 2) % 4, br)

        @pl.when(i + 4 < NCHUNK)
        def _():
            _idx_start(i + 4, bi)

    for k in range(4):
        _idx_start(k, k)
    _idx_wait(0)
    _gather(0, 0)
    _idx_wait(1)
    _gather(1, 1)

    def _outer(io, _):
        for b in range(4):
            _step(io * 4 + b, b, b % 2)
        return 0

    lax.fori_loop(0, (NCHUNK - 1) // 4, _outer, 0)
    _step(NCHUNK - 1, (NCHUNK - 1) % 4, (NCHUNK - 1) % 2)
    plsc.subcore_barrier()

    pltpu.sync_copy(dsh.at[pl.ds(nbase, ROWS_PER_TILE)],
                    denom_hbm.at[cid, pl.ds(nbase, ROWS_PER_TILE)])
    pltpu.sync_copy(nsh.at[pl.ds(nbase, ROWS_PER_TILE)],
                    numer_hbm.at[cid, pl.ds(nbase, ROWS_PER_TILE)])


@functools.partial(
    pl.kernel,
    out_type=[
        jax.ShapeDtypeStruct((NCORES, NP, D), jnp.float32),
        jax.ShapeDtypeStruct((NCORES, NP), jnp.float32),
    ],
    mesh=plsc.VectorSubcoreMesh(core_axis_name="c", subcore_axis_name="s"),
    compiler_params=pltpu.CompilerParams(needs_layout_passes=False),
    scratch_types=[
        pltpu.VMEM((N,), jnp.float32),
        pltpu.VMEM((N,), jnp.float32),
        pltpu.VMEM((16,), jnp.float32),
        pltpu.VMEM((2, C), jnp.int32),
        pltpu.VMEM((2, C), jnp.int32),
        pltpu.VMEM((2, C), jnp.int32),
        pltpu.VMEM((2, C), jnp.int32),
        pltpu.VMEM((C,), jnp.float32),
        pltpu.VMEM((C, D), jnp.float32),
        pltpu.VMEM((C,), jnp.float32),
        pltpu.VMEM((C, D), jnp.float32),
        pltpu.VMEM((ROWS_PER_TILE,), jnp.float32),
        pltpu.VMEM_SHARED((NP, D), jnp.float32),
        pltpu.VMEM_SHARED((NP,), jnp.float32),
        pltpu.SemaphoreType.DMA,
        pltpu.SemaphoreType.DMA,
        pltpu.SemaphoreType.DMA,
        pltpu.SemaphoreType.DMA,
        pltpu.SemaphoreType.DMA,
        pltpu.SemaphoreType.DMA,
    ],
)
def _edge_sc(a_hbm, b_hbm, bm_hbm, v_hbm, ed_hbm,
             numer_hbm, denom_hbm,
             at, bt, bmv, idxp0, idxp1, idxp2, idxp3,
             exb0, rows0, exb1, rows1,
             acc, nsh, dsh, semi0, semi1, semi2, semi3, semg0, semg1):
    _sc_body(a_hbm, b_hbm, bm_hbm, v_hbm, ed_hbm,
             numer_hbm, denom_hbm,
             at, bt, bmv, idxp0, idxp1, idxp2, idxp3,
             exb0, rows0, exb1, rows1,
             acc, nsh, dsh, semi0, semi1, semi2, semi3, semg0, semg1)


def _post_body(n0_ref, n1_ref, d0_ref, d1_ref, xh_ref, wp_ref, z_ref):
    agg = n0_ref[...] + n1_ref[...]
    d = d0_ref[...] + d1_ref[...]
    inv = jnp.where(d > 0.0, 1.0 / d, 0.0)
    out = agg * inv
    col = lax.broadcasted_iota(jnp.int32, out.shape, 1)
    out = out + jnp.where(col == 0, 1.0, 0.0)
    out = out / _norm_factor(out)
    op = jnp.dot(out, wp_ref[...].T, preferred_element_type=jnp.float32)
    op = op / _norm_factor(op)
    s = op + xh_ref[...]
    z_ref[...] = s / _norm_factor(s)


def _post(n0, n1, d0, d1, x_H, W_p):
    blk = 1000
    grid = N // blk
    return pl.pallas_call(
        _post_body,
        grid=(grid,),
        in_specs=[
            pl.BlockSpec((blk, D), lambda i: (i, 0)),
            pl.BlockSpec((blk, D), lambda i: (i, 0)),
            pl.BlockSpec((blk, 1), lambda i: (i, 0)),
            pl.BlockSpec((blk, 1), lambda i: (i, 0)),
            pl.BlockSpec((blk, D), lambda i: (i, 0)),
            pl.BlockSpec((D, D), lambda i: (0, 0)),
        ],
        out_specs=pl.BlockSpec((blk, D), lambda i: (i, 0)),
        out_shape=jax.ShapeDtypeStruct((N, D), jnp.float32),
    )(n0, n1, d0, d1, x_H, W_p)


@jax.jit
def kernel(x_H, x_S, edge_index, W_q, W_k, W_v, W_s, W_p):
    src = edge_index[0].astype(jnp.int32)
    dst = edge_index[1].astype(jnp.int32)
    ed = jnp.stack([src.reshape(E // C, C), dst.reshape(E // C, C)], axis=1)
    a2, b2, v, bm = _pre(x_S, x_H, W_q, W_k, W_v, W_s)
    bvec = jnp.broadcast_to(bm[0], (16,))
    numer, denom = _edge_sc(a2[:, 0], b2[:, 0], bvec, v, ed)
    z = _post(numer[0, :N], numer[1, :N],
              denom[0, :N, None], denom[1, :N, None], x_H, W_p)
    return z
